# Initial kernel scaffold; baseline (speedup 1.0000x reference)
import jax
import jax.numpy as jnp
from jax.experimental import pallas as pl


def kernel(image, flow):
    raise NotImplementedError("write your pallas kernel here")


# speedup vs baseline: 1.0132x; 1.0132x over previous
import functools

import jax
import jax.numpy as jnp
from jax import lax
from jax.experimental import pallas as pl
from jax.experimental.pallas import tpu as pltpu
from jax.experimental.pallas import tpu_sc as plsc

_B, _H, _W, _C = 2, 384, 384, 96
_NP = _B * _H * _W
_NW = 32
_K = 128
_CPR = _W // _K
_CPW = (_B * _H // _NW) * _CPR
_G = _K // 16


def _warp_kernel(table, fy_hbm, fx_hbm, out_hbm,
                 fy_v, fx_v, itl, itr, ibl, ibr, ax_v, ay_v,
                 tl, tr, bl, br, ob, sem):
  wid = lax.axis_index("s") * 2 + lax.axis_index("c")

  def chunk_body(t, carry):
    cid = wid * _CPW + t
    r = cid // _CPR
    xbase = (cid % _CPR) * _K
    y = r % _H
    b = r // _H
    p0 = cid * _K
    pltpu.sync_copy(fy_hbm.at[pl.ds(p0, _K)], fy_v)
    pltpu.sync_copy(fx_hbm.at[pl.ds(p0, _K)], fx_v)
    yf = y.astype(jnp.float32)
    boff = b * (_H * _W)
    for g in range(_G):
      sl = pl.ds(g * 16, 16)
      xf = (xbase + g * 16 + lax.iota(jnp.int32, 16)).astype(jnp.float32)
      qx = xf - fx_v[sl]
      qy = yf - fy_v[sl]
      qxc = jnp.minimum(jnp.maximum(qx, -1.0), jnp.float32(_W))
      qyc = jnp.minimum(jnp.maximum(qy, -1.0), jnp.float32(_H))
      fxi = jnp.minimum(jnp.maximum(qxc.astype(jnp.int32), 0), _W - 2)
      fyi = jnp.minimum(jnp.maximum(qyc.astype(jnp.int32), 0), _H - 2)
      axw = jnp.minimum(jnp.maximum(qx - fxi.astype(jnp.float32), 0.0), 1.0)
      ayw = jnp.minimum(jnp.maximum(qy - fyi.astype(jnp.float32), 0.0), 1.0)
      base = boff + fyi * _W + fxi
      itl[sl] = base
      itr[sl] = base + 1
      ibl[sl] = base + _W
      ibr[sl] = base + _W + 1
      ax_v[sl] = axw
      ay_v[sl] = ayw
    cps = [pltpu.async_copy(table.at[itl], tl, sem),
           pltpu.async_copy(table.at[itr], tr, sem),
           pltpu.async_copy(table.at[ibl], bl, sem),
           pltpu.async_copy(table.at[ibr], br, sem)]
    for cp in cps:
      cp.wait()

    def px_body(i, c2):
      isp = jnp.full((16,), i, jnp.int32)
      axs = plsc.load_gather(ax_v, [isp])
      ays = plsc.load_gather(ay_v, [isp])
      for c in range(_C // 16):
        csl = pl.ds(c * 16, 16)
        vtl = tl[i, csl]
        vtr = tr[i, csl]
        vbl = bl[i, csl]
        vbr = br[i, csl]
        top = axs * (vtr - vtl) + vtl
        bot = axs * (vbr - vbl) + vbl
        ob[i, csl] = ays * (bot - top) + top
      return c2

    lax.fori_loop(0, _K, px_body, 0)
    pltpu.sync_copy(ob, out_hbm.at[pl.ds(p0, _K)])
    return carry

  lax.fori_loop(0, _CPW, chunk_body, 0)


_warp = functools.partial(
    pl.kernel,
    out_type=jax.ShapeDtypeStruct((_NP, _C), jnp.float32),
    mesh=plsc.VectorSubcoreMesh(core_axis_name="c", subcore_axis_name="s"),
    compiler_params=pltpu.CompilerParams(
        needs_layout_passes=False, use_tc_tiling_on_sc=False),
    scratch_types=[
        pltpu.VMEM((_K,), jnp.float32),
        pltpu.VMEM((_K,), jnp.float32),
        pltpu.VMEM((_K,), jnp.int32),
        pltpu.VMEM((_K,), jnp.int32),
        pltpu.VMEM((_K,), jnp.int32),
        pltpu.VMEM((_K,), jnp.int32),
        pltpu.VMEM((_K,), jnp.float32),
        pltpu.VMEM((_K,), jnp.float32),
        pltpu.VMEM((_K, _C), jnp.float32),
        pltpu.VMEM((_K, _C), jnp.float32),
        pltpu.VMEM((_K, _C), jnp.float32),
        pltpu.VMEM((_K, _C), jnp.float32),
        pltpu.VMEM((_K, _C), jnp.float32),
        pltpu.SemaphoreType.DMA,
    ],
)(_warp_kernel)


def kernel(image, flow):
  table = image.reshape(_NP, _C)
  fy = flow[..., 0].reshape(_NP)
  fx = flow[..., 1].reshape(_NP)
  out = _warp(table, fy, fx)
  return out.reshape(_B, _H, _W, _C)


# speedup vs baseline: 1.4140x; 1.3956x over previous
import functools

import jax
import jax.numpy as jnp
from jax import lax
from jax.experimental import pallas as pl
from jax.experimental.pallas import tpu as pltpu
from jax.experimental.pallas import tpu_sc as plsc

_B, _H, _W, _C = 2, 384, 384, 96
_NP = _B * _H * _W
_NW = 32
_K = 96
_CPR = _W // _K
_CPW = (_B * _H // _NW) * _CPR
_G = _K // 16
_NC = _C // 16


def _warp_kernel(table, flow2, out_hbm, *refs):
  (fl0, fl1,
   itl0, itr0, ibl0, ibr0, itl1, itr1, ibl1, ibr1,
   ax0, ay0, ax1, ay1,
   tl0, tr0, bl0, br0, tl1, tr1, bl1, br1,
   ob0, ob1, sem_g, sem_o) = refs
  flow_v = (fl0, fl1)
  idxs = ((itl0, itr0, ibl0, ibr0), (itl1, itr1, ibl1, ibr1))
  bufs = ((tl0, tr0, bl0, br0), (tl1, tr1, bl1, br1))
  axr = (ax0, ax1)
  ayr = (ay0, ay1)
  obs = (ob0, ob1)

  wid = lax.axis_index("s") * 2 + lax.axis_index("c")
  c0 = wid * _CPW

  def setup(t, b):
    cid = c0 + t
    r = cid // _CPR
    xbase = (cid % _CPR) * _K
    y = r % _H
    boff = (r // _H) * (_H * _W)
    p0 = cid * _K
    pltpu.sync_copy(flow2.at[pl.ds(p0, _K)], flow_v[b])
    yf = y.astype(jnp.float32)
    for g in range(_G):
      sl = pl.ds(g * 16, 16)
      pix = lax.iota(jnp.int32, 16) + g * 16
      fyv = plsc.load_gather(flow_v[b], [pix, jnp.zeros((16,), jnp.int32)])
      fxv = plsc.load_gather(flow_v[b], [pix, jnp.ones((16,), jnp.int32)])
      xf = (xbase + g * 16 + lax.iota(jnp.int32, 16)).astype(jnp.float32)
      qx = xf - fxv
      qy = yf - fyv
      qxc = jnp.minimum(jnp.maximum(qx, -1.0), jnp.float32(_W))
      qyc = jnp.minimum(jnp.maximum(qy, -1.0), jnp.float32(_H))
      fxi = jnp.minimum(jnp.maximum(qxc.astype(jnp.int32), 0), _W - 2)
      fyi = jnp.minimum(jnp.maximum(qyc.astype(jnp.int32), 0), _H - 2)
      axw = jnp.minimum(jnp.maximum(qx - fxi.astype(jnp.float32), 0.0), 1.0)
      ayw = jnp.minimum(jnp.maximum(qy - fyi.astype(jnp.float32), 0.0), 1.0)
      base = boff + fyi * _W + fxi
      idxs[b][0][sl] = base
      idxs[b][1][sl] = base + 1
      idxs[b][2][sl] = base + _W
      idxs[b][3][sl] = base + _W + 1
      axr[b][sl] = axw
      ayr[b][sl] = ayw
    for i in range(4):
      pltpu.async_copy(table.at[idxs[b][i]], bufs[b][i], sem_g)

  setup(0, 0)

  @pl.loop(0, _CPW, step=2)
  def _chunk_pair(t2):
    for b in (0, 1):
      t = t2 + b
      for i in range(4):
        pltpu.make_async_copy(table.at[idxs[b][i]], bufs[b][i], sem_g).wait()

      @pl.when(t + 1 < _CPW)
      def _():
        setup(t + 1, 1 - b)

      @pl.when(t >= 2)
      def _():
        pltpu.make_async_copy(obs[b], out_hbm.at[pl.ds(0, _K)], sem_o).wait()

      tlb, trb, blb, brb = bufs[b]
      ob = obs[b]

      @pl.loop(0, _G)
      def _blend_group(g):
        axg = axr[b][pl.ds(g * 16, 16)]
        ayg = ayr[b][pl.ds(g * 16, 16)]
        for l in range(16):
          i = g * 16 + l
          axs = jnp.broadcast_to(axg[l], (16,))
          ays = jnp.broadcast_to(ayg[l], (16,))
          for c in range(_NC):
            csl = pl.ds(c * 16, 16)
            vtl = tlb[i, csl]
            vtr = trb[i, csl]
            vbl = blb[i, csl]
            vbr = brb[i, csl]
            top = axs * (vtr - vtl) + vtl
            bot = axs * (vbr - vbl) + vbl
            ob[i, csl] = ays * (bot - top) + top

      p0 = (c0 + t) * _K
      pltpu.async_copy(ob, out_hbm.at[pl.ds(p0, _K)], sem_o)

  for _ in range(2):
    pltpu.make_async_copy(obs[0], out_hbm.at[pl.ds(0, _K)], sem_o).wait()


_warp = functools.partial(
    pl.kernel,
    out_type=jax.ShapeDtypeStruct((_NP, _C), jnp.float32),
    mesh=plsc.VectorSubcoreMesh(core_axis_name="c", subcore_axis_name="s"),
    compiler_params=pltpu.CompilerParams(
        needs_layout_passes=False, use_tc_tiling_on_sc=False),
    scratch_types=(
        [pltpu.VMEM((_K, 2), jnp.float32)] * 2
        + [pltpu.VMEM((_K,), jnp.int32)] * 8
        + [pltpu.VMEM((_K,), jnp.float32)] * 4
        + [pltpu.VMEM((_K, _C), jnp.float32)] * 8
        + [pltpu.VMEM((_K, _C), jnp.float32)] * 2
        + [pltpu.SemaphoreType.DMA] * 2
    ),
)(_warp_kernel)


def kernel(image, flow):
  table = image.reshape(_NP, _C)
  flow2 = flow.reshape(_NP, 2)
  out = _warp(table, flow2)
  return out.reshape(_B, _H, _W, _C)


# speedup vs baseline: 1.7122x; 1.2108x over previous
import functools

import jax
import jax.numpy as jnp
from jax import lax
from jax.experimental import pallas as pl
from jax.experimental.pallas import tpu as pltpu
from jax.experimental.pallas import tpu_sc as plsc

_B, _H, _W, _C = 2, 384, 384, 96
_D = 128
_NP = _B * _H * _W
_NW = 32
_K = 64
_CPR = _W // _K
_CPW = (_B * _H // _NW) * _CPR
_G = _K // 16
_NC = _C // 16


def _warp_kernel(table, flw, out_hbm, *refs):
  (fl0, fl1,
   itl0, itr0, ibl0, ibr0, itl1, itr1, ibl1, ibr1,
   ax0, ay0, ax1, ay1,
   tl0, tr0, bl0, br0, tl1, tr1, bl1, br1,
   ob0, ob1, sem_g, sem_o) = refs
  flow_v = (fl0, fl1)
  idxs = ((itl0, itr0, ibl0, ibr0), (itl1, itr1, ibl1, ibr1))
  bufs = ((tl0, tr0, bl0, br0), (tl1, tr1, bl1, br1))
  axr = (ax0, ax1)
  ayr = (ay0, ay1)
  obs = (ob0, ob1)

  wid = lax.axis_index("s") * 2 + lax.axis_index("c")
  c0 = wid * _CPW

  zv = jnp.zeros((16,), jnp.float32)
  for ob in obs:
    @pl.loop(0, _K)
    def _zero_pad(i):
      ob[i, pl.ds(_C, 16)] = zv
      ob[i, pl.ds(_C + 16, 16)] = zv

  def setup(t, b):
    cid = c0 + t
    r = cid // _CPR
    xbase = (cid % _CPR) * _K
    y = r % _H
    boff = (r // _H) * (_H * _W)
    pltpu.sync_copy(flw.at[cid], flow_v[b])
    yf = y.astype(jnp.float32)
    for g in range(_G):
      sl = pl.ds(g * 16, 16)
      pix = lax.iota(jnp.int32, 16) + g * 16
      fyv = plsc.load_gather(flow_v[b], [pix * 2])
      fxv = plsc.load_gather(flow_v[b], [pix * 2 + 1])
      xf = (xbase + g * 16 + lax.iota(jnp.int32, 16)).astype(jnp.float32)
      qx = xf - fxv
      qy = yf - fyv
      qxc = jnp.minimum(jnp.maximum(qx, -1.0), jnp.float32(_W))
      qyc = jnp.minimum(jnp.maximum(qy, -1.0), jnp.float32(_H))
      fxi = jnp.minimum(jnp.maximum(qxc.astype(jnp.int32), 0), _W - 2)
      fyi = jnp.minimum(jnp.maximum(qyc.astype(jnp.int32), 0), _H - 2)
      axw = jnp.minimum(jnp.maximum(qx - fxi.astype(jnp.float32), 0.0), 1.0)
      ayw = jnp.minimum(jnp.maximum(qy - fyi.astype(jnp.float32), 0.0), 1.0)
      base = boff + fyi * _W + fxi
      idxs[b][0][sl] = base
      idxs[b][1][sl] = base + 1
      idxs[b][2][sl] = base + _W
      idxs[b][3][sl] = base + _W + 1
      axr[b][sl] = axw
      ayr[b][sl] = ayw
    for i in range(4):
      pltpu.async_copy(table.at[idxs[b][i]], bufs[b][i], sem_g)

  setup(0, 0)

  @pl.loop(0, _CPW, step=2)
  def _chunk_pair(t2):
    for b in (0, 1):
      t = t2 + b
      for i in range(4):
        pltpu.make_async_copy(table.at[idxs[b][i]], bufs[b][i], sem_g).wait()

      @pl.when(t + 1 < _CPW)
      def _():
        setup(t + 1, 1 - b)

      @pl.when(t >= 2)
      def _():
        pltpu.make_async_copy(obs[b], out_hbm.at[pl.ds(0, _K)], sem_o).wait()

      tlb, trb, blb, brb = bufs[b]
      ob = obs[b]

      @pl.loop(0, _G)
      def _blend_group(g):
        axg = axr[b][pl.ds(g * 16, 16)]
        ayg = ayr[b][pl.ds(g * 16, 16)]
        for l in range(16):
          i = g * 16 + l
          axs = jnp.broadcast_to(axg[l], (16,))
          ays = jnp.broadcast_to(ayg[l], (16,))
          for c in range(_NC):
            csl = pl.ds(c * 16, 16)
            vtl = tlb[i, csl]
            vtr = trb[i, csl]
            vbl = blb[i, csl]
            vbr = brb[i, csl]
            top = axs * (vtr - vtl) + vtl
            bot = axs * (vbr - vbl) + vbl
            ob[i, csl] = ays * (bot - top) + top

      p0 = (c0 + t) * _K
      pltpu.async_copy(ob, out_hbm.at[pl.ds(p0, _K)], sem_o)

  for _ in range(2):
    pltpu.make_async_copy(obs[0], out_hbm.at[pl.ds(0, _K)], sem_o).wait()


_warp = functools.partial(
    pl.kernel,
    out_type=jax.ShapeDtypeStruct((_NP, _D), jnp.float32),
    mesh=plsc.VectorSubcoreMesh(core_axis_name="c", subcore_axis_name="s"),
    compiler_params=pltpu.CompilerParams(
        needs_layout_passes=False, use_tc_tiling_on_sc=True),
    scratch_types=(
        [pltpu.VMEM((2 * _K,), jnp.float32)] * 2
        + [pltpu.VMEM((_K,), jnp.int32)] * 8
        + [pltpu.VMEM((_K,), jnp.float32)] * 4
        + [pltpu.VMEM((_K, _D), jnp.float32)] * 8
        + [pltpu.VMEM((_K, _D), jnp.float32)] * 2
        + [pltpu.SemaphoreType.DMA] * 2
    ),
)(_warp_kernel)


def kernel(image, flow):
  table = jnp.pad(image.reshape(_NP, _C), ((0, 0), (0, _D - _C)))
  flw = flow.reshape(_NP * 2 // _D, _D)
  out = _warp(table, flw)
  return out[:, :_C].reshape(_B, _H, _W, _C)


# speedup vs baseline: 1.9040x; 1.1121x over previous
import functools

import jax
import jax.numpy as jnp
from jax import lax
from jax.experimental import pallas as pl
from jax.experimental.pallas import tpu as pltpu
from jax.experimental.pallas import tpu_sc as plsc

_B, _H, _W, _C = 2, 384, 384, 96
_D = 128
_NP = _B * _H * _W
_NW = 32
_K = 64
_CPR = _W // _K
_CPW = (_B * _H // _NW) * _CPR
_G = _K // 16
_NC = _C // 16


def _warp_kernel(table, fyr, fxr, out_hbm, *refs):
  (fy0, fy1, fx0, fx1,
   itl0, itr0, ibl0, ibr0, itl1, itr1, ibl1, ibr1,
   ax0, ay0, ax1, ay1,
   tl0, tr0, bl0, br0, tl1, tr1, bl1, br1,
   ob0, ob1, sem_g, sem_o) = refs
  fyv = (fy0, fy1)
  fxv = (fx0, fx1)
  idxs = ((itl0, itr0, ibl0, ibr0), (itl1, itr1, ibl1, ibr1))
  bufs = ((tl0, tr0, bl0, br0), (tl1, tr1, bl1, br1))
  axr = (ax0, ax1)
  ayr = (ay0, ay1)
  obs = (ob0, ob1)

  wid = lax.axis_index("s") * 2 + lax.axis_index("c")
  c0 = wid * _CPW

  def setup(t, b):
    cid = c0 + t
    r = cid // _CPR
    xbase = (cid % _CPR) * _K
    y = r % _H
    boff = (r // _H) * (_H * _W)
    frow = cid // 2
    foff = (cid % 2) * _K
    pltpu.sync_copy(fyr.at[frow, pl.ds(foff, _K)], fyv[b])
    pltpu.sync_copy(fxr.at[frow, pl.ds(foff, _K)], fxv[b])
    yf = y.astype(jnp.float32)
    for g in range(_G):
      sl = pl.ds(g * 16, 16)
      qx = (xbase + g * 16 + lax.iota(jnp.int32, 16)).astype(jnp.float32) \
          - fxv[b][sl]
      qy = yf - fyv[b][sl]
      qxc = jnp.minimum(jnp.maximum(qx, -1.0), jnp.float32(_W))
      qyc = jnp.minimum(jnp.maximum(qy, -1.0), jnp.float32(_H))
      fxi = jnp.minimum(jnp.maximum(qxc.astype(jnp.int32), 0), _W - 2)
      fyi = jnp.minimum(jnp.maximum(qyc.astype(jnp.int32), 0), _H - 2)
      axw = jnp.minimum(jnp.maximum(qx - fxi.astype(jnp.float32), 0.0), 1.0)
      ayw = jnp.minimum(jnp.maximum(qy - fyi.astype(jnp.float32), 0.0), 1.0)
      base = boff + fyi * _W + fxi
      idxs[b][0][sl] = base
      idxs[b][1][sl] = base + 1
      idxs[b][2][sl] = base + _W
      idxs[b][3][sl] = base + _W + 1
      axr[b][sl] = axw
      ayr[b][sl] = ayw
    for i in range(4):
      pltpu.async_copy(table.at[idxs[b][i]], bufs[b][i], sem_g)

  setup(0, 0)

  @pl.loop(0, _CPW, step=2)
  def _chunk_pair(t2):
    for b in (0, 1):
      t = t2 + b
      for i in range(4):
        pltpu.make_async_copy(table.at[idxs[b][i]], bufs[b][i], sem_g).wait()

      @pl.when(t + 1 < _CPW)
      def _():
        setup(t + 1, 1 - b)

      @pl.when(t >= 2)
      def _():
        pltpu.make_async_copy(
            obs[b], out_hbm.at[0, 0, pl.ds(0, _K), :], sem_o).wait()

      tlb, trb, blb, brb = bufs[b]
      ob = obs[b]

      @pl.loop(0, _G)
      def _blend_group(g):
        axg = axr[b][pl.ds(g * 16, 16)]
        ayg = ayr[b][pl.ds(g * 16, 16)]
        for l in range(16):
          i = g * 16 + l
          axs = jnp.broadcast_to(axg[l], (16,))
          ays = jnp.broadcast_to(ayg[l], (16,))
          for c in range(_NC):
            csl = pl.ds(c * 16, 16)
            vtl = tlb[i, csl]
            vtr = trb[i, csl]
            vbl = blb[i, csl]
            vbr = brb[i, csl]
            top = axs * (vtr - vtl) + vtl
            bot = axs * (vbr - vbl) + vbl
            ob[i, csl] = ays * (bot - top) + top

      cid = c0 + t
      r = cid // _CPR
      pltpu.async_copy(
          ob,
          out_hbm.at[r // _H, r % _H, pl.ds((cid % _CPR) * _K, _K), :],
          sem_o)

  for _ in range(2):
    pltpu.make_async_copy(
        obs[0], out_hbm.at[0, 0, pl.ds(0, _K), :], sem_o).wait()


_warp = functools.partial(
    pl.kernel,
    out_type=jax.ShapeDtypeStruct((_B, _H, _W, _C), jnp.float32),
    mesh=plsc.VectorSubcoreMesh(core_axis_name="c", subcore_axis_name="s"),
    compiler_params=pltpu.CompilerParams(
        needs_layout_passes=False, use_tc_tiling_on_sc=True),
    scratch_types=(
        [pltpu.VMEM((_K,), jnp.float32)] * 4
        + [pltpu.VMEM((_K,), jnp.int32)] * 8
        + [pltpu.VMEM((_K,), jnp.float32)] * 4
        + [pltpu.VMEM((_K, _D), jnp.float32)] * 8
        + [pltpu.VMEM((_K, _C), jnp.float32)] * 2
        + [pltpu.SemaphoreType.DMA] * 2
    ),
)(_warp_kernel)


def kernel(image, flow):
  table = jnp.pad(image, ((0, 0), (0, 0), (0, 0), (0, _D - _C)))
  table = table.reshape(_NP, _D)
  fyr = flow[..., 0].reshape(_NP // _D, _D)
  fxr = flow[..., 1].reshape(_NP // _D, _D)
  return _warp(table, fyr, fxr)
